# Initial kernel scaffold; baseline (speedup 1.0000x reference)
import jax
import jax.numpy as jnp
from jax.experimental import pallas as pl


def kernel(x_in, inds_perm):
    raise NotImplementedError("write your pallas kernel here")


# speedup vs baseline: 1.2407x; 1.2407x over previous
import functools

import jax
import jax.numpy as jnp
from jax import lax
from jax.experimental import pallas as pl
from jax.experimental.pallas import tpu as pltpu
from jax.experimental.pallas import tpu_sc as plsc

L = 16
NC = 2
NS = 16
NW = NC * NS
D = 128
R = 64 * 8192
ROWS_PER_W = R // NW
CH = 128
NCH = ROWS_PER_W // CH
CHW = CH * D


def _sc_permute(x_flat, perm):
    mesh = plsc.VectorSubcoreMesh(core_axis_name="c", subcore_axis_name="s")

    @functools.partial(
        pl.kernel,
        out_type=jax.ShapeDtypeStruct((R * D,), jnp.float32),
        mesh=mesh,
        compiler_params=pltpu.CompilerParams(needs_layout_passes=False),
        scratch_types=[
            pltpu.VMEM((D,), jnp.int32),
            pltpu.VMEM((CHW,), jnp.float32),
            pltpu.VMEM((CHW,), jnp.float32),
            pltpu.VMEM((CHW,), jnp.float32),
            pltpu.VMEM((CHW,), jnp.float32),
            pltpu.SemaphoreType.DMA,
            pltpu.SemaphoreType.DMA,
            pltpu.SemaphoreType.DMA,
            pltpu.SemaphoreType.DMA,
        ],
    )
    def k(x_hbm, perm_hbm, out_hbm, perm_v, in0, in1, out0, out1,
          in_sem0, in_sem1, out_sem0, out_sem1):
        wid = lax.axis_index("s") * NC + lax.axis_index("c")
        base = wid * (ROWS_PER_W * D)
        pltpu.sync_copy(perm_hbm, perm_v)

        ins = (in0, in1)
        outs = (out0, out1)
        in_sems = (in_sem0, in_sem1)
        out_sems = (out_sem0, out_sem1)

        def in_copy(chunk, b):
            return pltpu.make_async_copy(
                x_hbm.at[pl.ds(base + chunk * CHW, CHW)], ins[b], in_sems[b])

        def out_copy(chunk, b):
            return pltpu.make_async_copy(
                outs[b], out_hbm.at[pl.ds(base + chunk * CHW, CHW)],
                out_sems[b])

        in_copy(0, 0).start()
        in_copy(1, 1).start()

        def compute(b):
            src = ins[b]
            dst = outs[b]
            for j in range(D // L):
                pj = perm_v[pl.ds(j * L, L)]

                def row_body(r, idx):
                    vals = plsc.load_gather(src, [idx])
                    dst[pl.ds(r * D + j * L, L)] = vals
                    return idx + D

                lax.fori_loop(0, CH, row_body, pj, unroll=4)

        def chunk_body(i, carry):
            for b in range(2):
                chunk = 2 * i + b
                in_copy(chunk, b).wait()

                @pl.when(chunk >= 2)
                def _wait_out():
                    out_copy(chunk - 2, b).wait()

                compute(b)

                @pl.when(chunk + 2 < NCH)
                def _next_in():
                    in_copy(chunk + 2, b).start()

                out_copy(chunk, b).start()
            return carry

        lax.fori_loop(0, NCH // 2, chunk_body, 0)
        out_copy(NCH - 2, 0).wait()
        out_copy(NCH - 1, 1).wait()

    return k(x_flat, perm)


def kernel(x_in, inds_perm):
    x_flat = x_in.reshape(-1)
    out = _sc_permute(x_flat, inds_perm)
    return (out.reshape(x_in.shape), 0.0)


# speedup vs baseline: 1.3320x; 1.0736x over previous
import functools

import jax
import jax.numpy as jnp
from jax import lax
from jax.experimental import pallas as pl
from jax.experimental.pallas import tpu as pltpu
from jax.experimental.pallas import tpu_sc as plsc

L = 16
NC = 2
NS = 16
NW = NC * NS
D = 128
R = 64 * 8192
ROWS_PER_W = R // NW
CH = 128
NCH = ROWS_PER_W // CH
CHW = CH * D


def _sc_permute(x_flat, perm):
    mesh = plsc.VectorSubcoreMesh(core_axis_name="c", subcore_axis_name="s")

    @functools.partial(
        pl.kernel,
        out_type=jax.ShapeDtypeStruct((R * D,), jnp.float32),
        mesh=mesh,
        compiler_params=pltpu.CompilerParams(needs_layout_passes=False),
        scratch_types=[
            pltpu.VMEM((D,), jnp.int32),
            pltpu.VMEM((CHW,), jnp.float32),
            pltpu.VMEM((CHW,), jnp.float32),
            pltpu.VMEM((CHW,), jnp.float32),
            pltpu.VMEM((CHW,), jnp.float32),
            pltpu.SemaphoreType.DMA,
            pltpu.SemaphoreType.DMA,
            pltpu.SemaphoreType.DMA,
            pltpu.SemaphoreType.DMA,
        ],
    )
    def k(x_hbm, perm_hbm, out_hbm, perm_v, in0, in1, out0, out1,
          in_sem0, in_sem1, out_sem0, out_sem1):
        wid = lax.axis_index("s") * NC + lax.axis_index("c")
        base = wid * (ROWS_PER_W * D)
        pltpu.sync_copy(perm_hbm, perm_v)

        ins = (in0, in1)
        outs = (out0, out1)
        in_sems = (in_sem0, in_sem1)
        out_sems = (out_sem0, out_sem1)

        def in_copy(chunk, b):
            return pltpu.make_async_copy(
                x_hbm.at[pl.ds(base + chunk * CHW, CHW)], ins[b], in_sems[b])

        def out_copy(chunk, b):
            return pltpu.make_async_copy(
                outs[b], out_hbm.at[pl.ds(base + chunk * CHW, CHW)],
                out_sems[b])

        in_copy(0, 0).start()
        in_copy(1, 1).start()

        pjs = [perm_v[pl.ds(j * L, L)] for j in range(D // L)]

        def compute(b):
            src = ins[b]
            dst = outs[b]

            def row_body(r, carry):
                base = r * D
                for j in range(D // L):
                    idx = pjs[j] + base
                    dst[pl.ds(base + j * L, L)] = plsc.load_gather(src, [idx])
                return carry

            lax.fori_loop(0, CH, row_body, 0, unroll=2)

        def chunk_body(i, carry):
            for b in range(2):
                chunk = 2 * i + b
                in_copy(chunk, b).wait()

                @pl.when(chunk >= 2)
                def _wait_out():
                    out_copy(chunk - 2, b).wait()

                compute(b)

                @pl.when(chunk + 2 < NCH)
                def _next_in():
                    in_copy(chunk + 2, b).start()

                out_copy(chunk, b).start()
            return carry

        lax.fori_loop(0, NCH // 2, chunk_body, 0)
        out_copy(NCH - 2, 0).wait()
        out_copy(NCH - 1, 1).wait()

    return k(x_flat, perm)


def kernel(x_in, inds_perm):
    x_flat = x_in.reshape(-1)
    out = _sc_permute(x_flat, inds_perm)
    return (out.reshape(x_in.shape), 0.0)


# speedup vs baseline: 3.2971x; 2.4754x over previous
import functools

import jax
import jax.numpy as jnp
from jax import lax
from jax.experimental import pallas as pl
from jax.experimental.pallas import tpu as pltpu
from jax.experimental.pallas import tpu_sc as plsc

L = 16
NC = 2
NS = 16
NW = NC * NS
D = 128
R = 64 * 8192
ROWS_PER_W = R // NW
CH = 128
NCH = ROWS_PER_W // CH
CHW = CH * D


def _sc_permute(x_flat, perm):
    mesh = plsc.VectorSubcoreMesh(core_axis_name="c", subcore_axis_name="s")

    @functools.partial(
        pl.kernel,
        out_type=jax.ShapeDtypeStruct((R * D,), jnp.float32),
        mesh=mesh,
        compiler_params=pltpu.CompilerParams(needs_layout_passes=False),
        scratch_types=[
            pltpu.VMEM((D,), jnp.int32),
            pltpu.VMEM((CHW,), jnp.float32),
            pltpu.VMEM((CHW,), jnp.float32),
            pltpu.VMEM((CHW,), jnp.float32),
            pltpu.VMEM((CHW,), jnp.float32),
            pltpu.SemaphoreType.DMA,
            pltpu.SemaphoreType.DMA,
            pltpu.SemaphoreType.DMA,
            pltpu.SemaphoreType.DMA,
        ],
    )
    def k(x_hbm, perm_hbm, out_hbm, perm_v, in0, in1, out0, out1,
          in_sem0, in_sem1, out_sem0, out_sem1):
        wid = lax.axis_index("s") * NC + lax.axis_index("c")
        base = wid * (ROWS_PER_W * D)
        pltpu.sync_copy(perm_hbm, perm_v)

        ins = (in0, in1)
        outs = (out0, out1)
        in_sems = (in_sem0, in_sem1)
        out_sems = (out_sem0, out_sem1)

        def in_copy(chunk, b):
            return pltpu.make_async_copy(
                x_hbm.at[pl.ds(base + chunk * CHW, CHW)], ins[b], in_sems[b])

        def out_copy(chunk, b):
            return pltpu.make_async_copy(
                outs[b], out_hbm.at[pl.ds(base + chunk * CHW, CHW)],
                out_sems[b])

        in_copy(0, 0).start()
        in_copy(1, 1).start()

        pjs = [perm_v[pl.ds(j * L, L)] for j in range(D // L)]

        def compute(b):
            src = ins[b]
            dst = outs[b]

            def row_body(r, carry):
                base = r * D
                vals = [plsc.load_gather(src, [pjs[j] + base])
                        for j in range(D // L)]
                for j in range(D // L):
                    dst[pl.ds(base + j * L, L)] = vals[j]
                return carry

            lax.fori_loop(0, CH, row_body, 0, unroll=2)

        def chunk_body(i, carry):
            for b in range(2):
                chunk = 2 * i + b
                in_copy(chunk, b).wait()

                @pl.when(chunk >= 2)
                def _wait_out():
                    out_copy(chunk - 2, b).wait()

                compute(b)

                @pl.when(chunk + 2 < NCH)
                def _next_in():
                    in_copy(chunk + 2, b).start()

                out_copy(chunk, b).start()
            return carry

        lax.fori_loop(0, NCH // 2, chunk_body, 0)
        out_copy(NCH - 2, 0).wait()
        out_copy(NCH - 1, 1).wait()

    return k(x_flat, perm)


def kernel(x_in, inds_perm):
    x_flat = x_in.reshape(-1)
    out = _sc_permute(x_flat, inds_perm)
    return (out.reshape(x_in.shape), 0.0)
